# baseline (speedup 1.0000x reference)
import jax
import jax.numpy as jnp
from jax.experimental import pallas as pl
from jax.experimental.pallas import tpu as pltpu

N = 1024
D_IN = 2
HID = 16
D_OUT = 12
LAYERS = 4
B = 8
F = B * HID


def _gnn_kernel(atm_ref, x0_ref, bdf_ref, bf_ref, bdg_ref, bg_ref,
                bdl_ref, blast_ref, out_ref, t_scr, acc_scr):
    f32 = jnp.float32
    hi = jax.lax.Precision.HIGHEST

    def dot(a, b):
        return jax.lax.dot_general(a, b, (((1,), (0,)), ((), ())),
                                   preferred_element_type=f32)

    r = jax.lax.broadcasted_iota(jnp.int32, (F, F), 0)
    c = jax.lax.broadcasted_iota(jnp.int32, (F, F), 1)
    eye_f = jnp.where(r == c, 1.0, 0.0).astype(f32)

    y0 = dot(x0_ref[:], bdf_ref[:]) + bf_ref[:]
    t_scr[:, :] = y0
    out = y0

    for i in range(LAYERS):
        acc_scr[:, :] = jnp.zeros((F, N), f32)

        def body(s, carry):
            y = t_scr[pl.ds(s, 1), :]
            arow = atm_ref[pl.ds(s, 1), :]
            y_t = jax.lax.dot_general(eye_f, y, (((1,), (1,)), ((), ())),
                                      preferred_element_type=f32,
                                      precision=hi)
            acc_scr[:, :] = acc_scr[:, :] + y_t * arow
            return carry

        jax.lax.fori_loop(0, N, body, 0)

        prop = jax.lax.dot_general(acc_scr[:, :], eye_f,
                                   (((0,), (0,)), ((), ())),
                                   preferred_element_type=f32,
                                   precision=hi)
        new = dot(prop, bdg_ref[i]) + bg_ref[i:i + 1] + t_scr[:, :]
        out = out + new
        t_scr[:, :] = new

    lr = jnp.where(out >= 0.0, out, 0.01 * out)
    out_ref[:] = jax.nn.sigmoid(dot(lr, bdl_ref[:]) + blast_ref[:])


def kernel(x, emb1, emb2, Wl1, bl1, Wl2, bl2, Wf, bf, Wg, bg, Wlast, blast):
    f32 = jnp.float32
    nv1 = jnp.tanh(emb1 @ Wl1.T + bl1)
    nv2 = jnp.tanh(emb2 @ Wl2.T + bl2)
    a = nv1 @ nv2.T - nv2 @ nv1.T
    adj = jax.nn.relu(jnp.tanh(a))
    adjm = jnp.where(adj > 0.2, adj, 0.0)

    x0 = jnp.transpose(x[:, :, :, -1], (2, 0, 1)).reshape(N, B * D_IN)
    eye_b = jnp.eye(B, dtype=f32)
    bdf = jnp.kron(eye_b, Wf.T.astype(f32))
    bdg = jnp.stack([jnp.kron(eye_b, Wg[i].T) for i in range(LAYERS)])
    bdl = jnp.kron(eye_b, Wlast.T.astype(f32))
    bfr = jnp.tile(bf, B)[None]
    bgr = jnp.tile(bg, (1, B))
    blr = jnp.tile(blast, B)[None]

    res = pl.pallas_call(
        _gnn_kernel,
        out_shape=jax.ShapeDtypeStruct((N, B * D_OUT), f32),
        scratch_shapes=[pltpu.VMEM((N, F), f32), pltpu.VMEM((F, N), f32)],
    )(adjm.T, x0, bdf, bfr, bdg, bgr, bdl, blr)

    return jnp.transpose(res.reshape(N, B, D_OUT), (1, 2, 0))[..., None]


# speedup vs baseline: 2.2386x; 2.2386x over previous
import jax
import jax.numpy as jnp
from jax.experimental import pallas as pl
from jax.experimental.pallas import tpu as pltpu

N = 1024
D_IN = 2
HID = 16
D_OUT = 12
LAYERS = 4
B = 8
F = B * HID


def _gnn_kernel(atm_ref, x0_ref, bdf_ref, bf_ref, bdg_ref, bg_ref,
                bdl_ref, blast_ref, out_ref, t_scr, acc_scr):
    f32 = jnp.float32
    hi = jax.lax.Precision.HIGHEST

    def dot(a, b):
        return jax.lax.dot_general(a, b, (((1,), (0,)), ((), ())),
                                   preferred_element_type=f32)

    r = jax.lax.broadcasted_iota(jnp.int32, (F, F), 0)
    c = jax.lax.broadcasted_iota(jnp.int32, (F, F), 1)
    eye_f = jnp.where(r == c, 1.0, 0.0).astype(f32)

    y0 = dot(x0_ref[:], bdf_ref[:]) + bf_ref[:]
    t_scr[:, :] = y0
    out = y0

    for i in range(LAYERS):
        acc_scr[:, :] = jnp.zeros((F, N), f32)

        def body(s, carry):
            y = t_scr[pl.ds(s, 1), :]
            arow = atm_ref[pl.ds(s, 1), :]
            y_t = jax.lax.dot_general(eye_f, y, (((1,), (1,)), ((), ())),
                                      preferred_element_type=f32,
                                      precision=hi)
            acc_scr[:, :] = acc_scr[:, :] + y_t * arow
            return carry

        jax.lax.fori_loop(0, N, body, 0, unroll=8)

        prop = jax.lax.dot_general(acc_scr[:, :], eye_f,
                                   (((0,), (0,)), ((), ())),
                                   preferred_element_type=f32,
                                   precision=hi)
        new = dot(prop, bdg_ref[i]) + bg_ref[i:i + 1] + t_scr[:, :]
        out = out + new
        t_scr[:, :] = new

    lr = jnp.where(out >= 0.0, out, 0.01 * out)
    out_ref[:] = jax.nn.sigmoid(dot(lr, bdl_ref[:]) + blast_ref[:])


def kernel(x, emb1, emb2, Wl1, bl1, Wl2, bl2, Wf, bf, Wg, bg, Wlast, blast):
    f32 = jnp.float32
    nv1 = jnp.tanh(emb1 @ Wl1.T + bl1)
    nv2 = jnp.tanh(emb2 @ Wl2.T + bl2)
    a = nv1 @ nv2.T - nv2 @ nv1.T
    adj = jax.nn.relu(jnp.tanh(a))
    adjm = jnp.where(adj > 0.2, adj, 0.0)

    x0 = jnp.transpose(x[:, :, :, -1], (2, 0, 1)).reshape(N, B * D_IN)
    eye_b = jnp.eye(B, dtype=f32)
    bdf = jnp.kron(eye_b, Wf.T.astype(f32))
    bdg = jnp.stack([jnp.kron(eye_b, Wg[i].T) for i in range(LAYERS)])
    bdl = jnp.kron(eye_b, Wlast.T.astype(f32))
    bfr = jnp.tile(bf, B)[None]
    bgr = jnp.tile(bg, (1, B))
    blr = jnp.tile(blast, B)[None]

    res = pl.pallas_call(
        _gnn_kernel,
        out_shape=jax.ShapeDtypeStruct((N, B * D_OUT), f32),
        scratch_shapes=[pltpu.VMEM((N, F), f32), pltpu.VMEM((F, N), f32)],
    )(adjm.T, x0, bdf, bfr, bdg, bgr, bdl, blr)

    return jnp.transpose(res.reshape(N, B, D_OUT), (1, 2, 0))[..., None]
